# baseline (speedup 1.0000x reference)
import jax
import jax.numpy as jnp
from jax.experimental import pallas as pl
from jax.experimental.pallas import tpu as pltpu

_RANK = 4
_SCALING = 8 / 4
_EPS = 1e-5

_N = 8192
_ROWS = 8192
_NCORES = 2
_BR = 128
_NBUF = 6
_OSLOTS = 6
_STEPS = _ROWS // (_NCORES * _BR)


def _outer(x_hbm, sa_ref, sb_ref, ha_ref, hb_ref, o_hbm, obuf, osem, cnt):
    core = pl.program_id(0)
    scale = jnp.sum(sa_ref[...] * sb_ref[...], axis=0, keepdims=True) * _SCALING
    shift = jnp.sum(ha_ref[...] * hb_ref[...], axis=0, keepdims=True) * _SCALING
    cnt[0] = 0

    def out_copy(slot, step):
        row0 = (core * _STEPS + step) * _BR
        return pltpu.make_async_copy(
            obuf.at[slot], o_hbm.at[pl.ds(row0, _BR), :], osem.at[slot]
        )

    def body(x_ref):
        j = cnt[0]
        slot = jax.lax.rem(j, _OSLOTS)

        @pl.when(j >= _OSLOTS)
        def _():
            out_copy(slot, j).wait()

        x = x_ref[...]
        mean = jnp.mean(x, axis=-1, keepdims=True)
        xc = x - mean
        var = jnp.mean(xc * xc, axis=-1, keepdims=True)
        obuf[slot] = xc * (jax.lax.rsqrt(var + _EPS) * scale) + shift
        out_copy(slot, j).start()
        cnt[0] = j + 1

    pipe = pltpu.emit_pipeline(
        body,
        grid=(_STEPS,),
        in_specs=[
            pl.BlockSpec(
                (_BR, _N),
                lambda j: (core * _STEPS + j, 0),
                pipeline_mode=pl.Buffered(buffer_count=_NBUF),
            )
        ],
        out_specs=[],
    )
    pipe(x_hbm)

    for k in range(_OSLOTS):
        out_copy(k, 0).wait()


def kernel(x, lora_scale_A, lora_scale_B, lora_shift_A, lora_shift_B):
    B, S, N = x.shape
    rows = B * S
    x2 = x.reshape(rows, N)
    sa = lora_scale_A.T
    ha = lora_shift_A.T

    lora_spec = pl.BlockSpec((_RANK, N), lambda i: (0, 0))
    out = pl.pallas_call(
        _outer,
        grid=(_NCORES,),
        in_specs=[
            pl.BlockSpec(memory_space=pl.ANY),
            lora_spec,
            lora_spec,
            lora_spec,
            lora_spec,
        ],
        out_specs=pl.BlockSpec(memory_space=pl.ANY),
        out_shape=jax.ShapeDtypeStruct((rows, N), x.dtype),
        scratch_shapes=[
            pltpu.VMEM((_OSLOTS, _BR, _N), jnp.float32),
            pltpu.SemaphoreType.DMA((_OSLOTS,)),
            pltpu.SMEM((1,), jnp.int32),
        ],
        compiler_params=pltpu.CompilerParams(
            dimension_semantics=("parallel",),
            vmem_limit_bytes=63 * 1024 * 1024,
        ),
    )(x2, sa, lora_scale_B, ha, lora_shift_B)
    return out.reshape(B, S, N)


# speedup vs baseline: 1.0011x; 1.0011x over previous
import jax
import jax.numpy as jnp
from jax.experimental import pallas as pl
from jax.experimental.pallas import tpu as pltpu

_RANK = 4
_SCALING = 8 / 4
_EPS = 1e-5

_N = 8192
_ROWS = 8192
_NCORES = 2
_BR = 128
_NBUF = 6
_OSLOTS = 4
_STEPS = _ROWS // (_NCORES * _BR)


def _outer(x_hbm, sa_ref, sb_ref, ha_ref, hb_ref, o_hbm, obuf, osem, cnt):
    core = pl.program_id(0)
    scale = jnp.sum(sa_ref[...] * sb_ref[...], axis=0, keepdims=True) * _SCALING
    shift = jnp.sum(ha_ref[...] * hb_ref[...], axis=0, keepdims=True) * _SCALING
    cnt[0] = 0

    def out_copy(slot, step):
        row0 = (core * _STEPS + step) * _BR
        return pltpu.make_async_copy(
            obuf.at[slot], o_hbm.at[pl.ds(row0, _BR), :], osem.at[slot]
        )

    def body(x_ref):
        j = cnt[0]
        slot = jax.lax.rem(j, _OSLOTS)

        @pl.when(j >= _OSLOTS)
        def _():
            out_copy(slot, j).wait()

        x = x_ref[...]
        mean = jnp.mean(x, axis=-1, keepdims=True)
        xc = x - mean
        var = jnp.mean(xc * xc, axis=-1, keepdims=True)
        obuf[slot] = xc * (jax.lax.rsqrt(var + _EPS) * scale) + shift
        out_copy(slot, j).start()
        cnt[0] = j + 1

    pipe = pltpu.emit_pipeline(
        body,
        grid=(_STEPS,),
        in_specs=[
            pl.BlockSpec(
                (_BR, _N),
                lambda j: (core * _STEPS + j, 0),
                pipeline_mode=pl.Buffered(buffer_count=_NBUF),
            )
        ],
        out_specs=[],
    )
    pipe(x_hbm)

    for k in range(_OSLOTS):
        out_copy(k, 0).wait()


def kernel(x, lora_scale_A, lora_scale_B, lora_shift_A, lora_shift_B):
    B, S, N = x.shape
    rows = B * S
    x2 = x.reshape(rows, N)
    sa = lora_scale_A.T
    ha = lora_shift_A.T

    lora_spec = pl.BlockSpec((_RANK, N), lambda i: (0, 0))
    out = pl.pallas_call(
        _outer,
        grid=(_NCORES,),
        in_specs=[
            pl.BlockSpec(memory_space=pl.ANY),
            lora_spec,
            lora_spec,
            lora_spec,
            lora_spec,
        ],
        out_specs=pl.BlockSpec(memory_space=pl.ANY),
        out_shape=jax.ShapeDtypeStruct((rows, N), x.dtype),
        scratch_shapes=[
            pltpu.VMEM((_OSLOTS, _BR, _N), jnp.float32),
            pltpu.SemaphoreType.DMA((_OSLOTS,)),
            pltpu.SMEM((1,), jnp.int32),
        ],
        compiler_params=pltpu.CompilerParams(
            dimension_semantics=("parallel",),
            vmem_limit_bytes=63 * 1024 * 1024,
        ),
    )(x2, sa, lora_scale_B, ha, lora_shift_B)
    return out.reshape(B, S, N)
